# baseline (speedup 1.0000x reference)
import functools

import jax
import jax.numpy as jnp
from jax import lax
from jax.experimental import pallas as pl
from jax.experimental.pallas import tpu as pltpu
from jax.experimental.pallas import tpu_sc as plsc

_NC = 2
_NS = 16
_NW = _NC * _NS
_MAXCHUNK = 320
_NBUF = 3
_LEAD = 2


def _schedule(b_req):
    sizes = []
    rem = b_req
    while rem > _MAXCHUNK + _MAXCHUNK // 2:
        sizes.append(_MAXCHUNK)
        rem -= _MAXCHUNK
    while rem > 0:
        s = min(rem, max(8, ((rem + 1) // 2 + 7) // 8 * 8))
        sizes.append(s)
        rem -= s
    return tuple(sizes)


@functools.partial(jax.jit, static_argnums=(2,))
def _sc_gather(table, idx, sizes):
    V, D = table.shape
    B = idx.shape[0]
    n_chunks = len(sizes)
    b_per_w = sum(sizes)
    prefix = [sum(sizes[:c]) for c in range(n_chunks)]
    nbuf = min(_NBUF, n_chunks)
    lead = min(_LEAD, nbuf - 1, n_chunks - 1)
    mesh = plsc.VectorSubcoreMesh(core_axis_name="c", subcore_axis_name="s")

    @functools.partial(
        pl.kernel,
        mesh=mesh,
        out_type=jax.ShapeDtypeStruct((B, D), jnp.float32),
        scratch_types=(
            [pltpu.VMEM((b_per_w,), jnp.int32)]
            + [pltpu.VMEM((max(sizes), D), jnp.float32) for _ in range(nbuf)]
            + [pltpu.SemaphoreType.DMA for _ in range(2 * nbuf)]
        ),
    )
    def gather_kernel(table_hbm, idx_hbm, out_hbm, idx_v, *rest):
        rbufs = rest[:nbuf]
        gsems = rest[nbuf:2 * nbuf]
        wsems = rest[2 * nbuf:]
        wid = lax.axis_index("s") * _NC + lax.axis_index("c")
        wbase = pl.multiple_of(jnp.minimum(wid * b_per_w, B - b_per_w), 8)
        starts = [pl.multiple_of(
            jnp.minimum(wbase + prefix[c], B - sizes[c]), 8)
            for c in range(n_chunks)]

        def fire_gather(c):
            off = pl.multiple_of(starts[c] - wbase, 8)
            return pltpu.async_copy(
                table_hbm.at[idx_v.at[pl.ds(off, sizes[c])]],
                rbufs[c % nbuf].at[pl.ds(0, sizes[c])], gsems[c % nbuf])

        gathers = [None] * n_chunks
        writes = [None] * n_chunks
        pltpu.sync_copy(idx_hbm.at[pl.ds(wbase, sizes[0])],
                        idx_v.at[pl.ds(0, sizes[0])])
        gathers[0] = fire_gather(0)
        if n_chunks > 1:
            pltpu.sync_copy(
                idx_hbm.at[pl.ds(wbase + sizes[0], b_per_w - sizes[0])],
                idx_v.at[pl.ds(sizes[0], b_per_w - sizes[0])])
        for c in range(1, lead):
            gathers[c] = fire_gather(c)
        for c in range(n_chunks):
            g = c + lead
            if g < n_chunks:
                if g >= nbuf:
                    writes[g - nbuf].wait()
                gathers[g] = fire_gather(g)
            gathers[c].wait()
            writes[c] = pltpu.async_copy(
                rbufs[c % nbuf].at[pl.ds(0, sizes[c])],
                out_hbm.at[pl.ds(starts[c], sizes[c])], wsems[c % nbuf])
        for c in range(max(0, n_chunks - nbuf), n_chunks):
            writes[c].wait()

    return gather_kernel(table, idx)


def kernel(this_level_g, this_level_h, idx):
    del this_level_g
    B = idx.shape[0]
    b_req = -(-B // (_NW * 8)) * 8
    return _sc_gather(this_level_h, idx.astype(jnp.int32), _schedule(b_req))


# speedup vs baseline: 1.0463x; 1.0463x over previous
import functools

import jax
import jax.numpy as jnp
from jax import lax
from jax.experimental import pallas as pl
from jax.experimental.pallas import tpu as pltpu
from jax.experimental.pallas import tpu_sc as plsc

_NC = 2
_NS = 16
_NW = _NC * _NS
_MAXCHUNK = 320
_NBUF = 3
_LEAD = 2


def _schedule(b_req):
    sizes = []
    rem = b_req
    while rem > _MAXCHUNK + _MAXCHUNK // 2:
        sizes.append(_MAXCHUNK)
        rem -= _MAXCHUNK
    if rem > 128:
        sizes.append(rem - 64)
        rem = 64
    if rem > 0:
        sizes.append(rem)
    return tuple(sizes)


@functools.partial(jax.jit, static_argnums=(2,))
def _sc_gather(table, idx, sizes):
    V, D = table.shape
    B = idx.shape[0]
    n_chunks = len(sizes)
    b_per_w = sum(sizes)
    prefix = [sum(sizes[:c]) for c in range(n_chunks)]
    nbuf = min(_NBUF, n_chunks)
    lead = min(_LEAD, nbuf - 1, n_chunks - 1)
    mesh = plsc.VectorSubcoreMesh(core_axis_name="c", subcore_axis_name="s")

    @functools.partial(
        pl.kernel,
        mesh=mesh,
        out_type=jax.ShapeDtypeStruct((B, D), jnp.float32),
        scratch_types=(
            [pltpu.VMEM((b_per_w,), jnp.int32)]
            + [pltpu.VMEM((max(sizes), D), jnp.float32) for _ in range(nbuf)]
            + [pltpu.SemaphoreType.DMA for _ in range(2 * nbuf)]
        ),
    )
    def gather_kernel(table_hbm, idx_hbm, out_hbm, idx_v, *rest):
        rbufs = rest[:nbuf]
        gsems = rest[nbuf:2 * nbuf]
        wsems = rest[2 * nbuf:]
        wid = lax.axis_index("s") * _NC + lax.axis_index("c")
        wbase = pl.multiple_of(jnp.minimum(wid * b_per_w, B - b_per_w), 8)
        starts = [pl.multiple_of(
            jnp.minimum(wbase + prefix[c], B - sizes[c]), 8)
            for c in range(n_chunks)]

        def fire_gather(c):
            off = pl.multiple_of(starts[c] - wbase, 8)
            return pltpu.async_copy(
                table_hbm.at[idx_v.at[pl.ds(off, sizes[c])]],
                rbufs[c % nbuf].at[pl.ds(0, sizes[c])], gsems[c % nbuf])

        gathers = [None] * n_chunks
        writes = [None] * n_chunks
        pltpu.sync_copy(idx_hbm.at[pl.ds(wbase, sizes[0])],
                        idx_v.at[pl.ds(0, sizes[0])])
        gathers[0] = fire_gather(0)
        if n_chunks > 1:
            pltpu.sync_copy(
                idx_hbm.at[pl.ds(wbase + sizes[0], b_per_w - sizes[0])],
                idx_v.at[pl.ds(sizes[0], b_per_w - sizes[0])])
        for c in range(1, lead):
            gathers[c] = fire_gather(c)
        for c in range(n_chunks):
            g = c + lead
            if g < n_chunks:
                if g >= nbuf:
                    writes[g - nbuf].wait()
                gathers[g] = fire_gather(g)
            gathers[c].wait()
            writes[c] = pltpu.async_copy(
                rbufs[c % nbuf].at[pl.ds(0, sizes[c])],
                out_hbm.at[pl.ds(starts[c], sizes[c])], wsems[c % nbuf])
        for c in range(max(0, n_chunks - nbuf), n_chunks):
            writes[c].wait()

    return gather_kernel(table, idx)


def kernel(this_level_g, this_level_h, idx):
    del this_level_g
    B = idx.shape[0]
    b_req = -(-B // (_NW * 8)) * 8
    return _sc_gather(this_level_h, idx.astype(jnp.int32), _schedule(b_req))
